# Initial kernel scaffold; baseline (speedup 1.0000x reference)
import jax
import jax.numpy as jnp
from jax.experimental import pallas as pl


def kernel(x, edge_index, edge_attr, Wp, bp, W0, b0, W1, b1, W2, b2, g0, be0, g1, be1, g2, be2):
    raise NotImplementedError("write your pallas kernel here")


# speedup vs baseline: 11.8597x; 11.8597x over previous
import functools
import jax
import jax.numpy as jnp
from jax import lax
from jax.experimental import pallas as pl
from jax.experimental.pallas import tpu as pltpu
from jax.experimental.pallas import tpu_sc as plsc

N = 10000
E = 320000
D = 128
EPS = 1e-5

NC = 2
NS = 16
NW = NC * NS

CH = 128
NCH = 79
E_W = CH * NCH
E_PAD = NW * E_W

N_PAD = 10240
ROWS_W = N_PAD // NS
BLK = 256
GRID = N_PAD // BLK

@functools.cache
def _mesh():
  return plsc.VectorSubcoreMesh(
      core_axis_name="c", subcore_axis_name="s", num_cores=NC, num_subcores=NS)


def _deg_body(dst_hbm, zeros128, ones128, out_hbm, dstb, onesv, acc, sem):
  c = lax.axis_index("c")
  s = lax.axis_index("s")
  w = c * NS + s
  pltpu.sync_copy(zeros128, acc.at[pl.ds(s * ROWS_W, ROWS_W)])
  pltpu.sync_copy(ones128, onesv)
  plsc.subcore_barrier()
  base = w * E_W

  def chunk(i, _):
    off = base + i * CH
    pltpu.sync_copy(dst_hbm.at[pl.ds(off, CH)], dstb.at[0])
    pltpu.sync_copy(onesv, acc.at[dstb.at[0]], add=True)
    return _

  lax.fori_loop(0, NCH, chunk, 0)
  plsc.subcore_barrier()
  pltpu.sync_copy(acc.at[pl.ds(s * ROWS_W, ROWS_W)],
                  out_hbm.at[c, pl.ds(s * ROWS_W, ROWS_W)])


@functools.cache
def _deg_kernel():
  return pl.kernel(
      _deg_body,
      out_type=jax.ShapeDtypeStruct((NC, N_PAD, D), jnp.float32),
      mesh=_mesh(),
      scratch_types=[
          pltpu.VMEM((1, CH), jnp.int32),
          pltpu.VMEM((CH, D), jnp.float32),
          pltpu.VMEM_SHARED((N_PAD, D), jnp.float32),
          pltpu.SemaphoreType.DMA,
      ],
  )


def _edge_body(y_hbm, src_hbm, dst_hbm, zeros128, out_hbm,
               srcb, dstb, rows, acc, sem):
  c = lax.axis_index("c")
  s = lax.axis_index("s")
  w = c * NS + s
  pltpu.sync_copy(zeros128, acc.at[pl.ds(s * ROWS_W, ROWS_W)])
  plsc.subcore_barrier()
  base = w * E_W

  def chunk(i, _):
    off = base + i * CH
    pltpu.sync_copy(src_hbm.at[pl.ds(off, CH)], srcb)
    pltpu.sync_copy(dst_hbm.at[pl.ds(off, CH)], dstb.at[0])
    pltpu.async_copy(y_hbm.at[srcb], rows, sem).wait()
    pltpu.sync_copy(rows, acc.at[dstb.at[0]], add=True)
    return _

  lax.fori_loop(0, NCH, chunk, 0)
  plsc.subcore_barrier()
  pltpu.sync_copy(acc.at[pl.ds(s * ROWS_W, ROWS_W)],
                  out_hbm.at[c, pl.ds(s * ROWS_W, ROWS_W)])


@functools.cache
def _edge_kernel():
  return pl.kernel(
      _edge_body,
      out_type=jax.ShapeDtypeStruct((NC, N_PAD, D), jnp.float32),
      mesh=_mesh(),
      scratch_types=[
          pltpu.VMEM((CH,), jnp.int32),
          pltpu.VMEM((1, CH), jnp.int32),
          pltpu.VMEM((CH, D), jnp.float32),
          pltpu.VMEM_SHARED((N_PAD, D), jnp.float32),
          pltpu.SemaphoreType.DMA,
      ],
  )


def _dot(a, b):
  return jax.lax.dot_general(a, b, (((1,), (0,)), ((), ())),
                             precision=lax.Precision.HIGHEST,
                             preferred_element_type=jnp.float32)


def _tc0_body(x_ref, wp_ref, bp_ref, w0_ref, degp_ref, h_ref, y0_ref, dis_ref):
  xb = x_ref[...]
  h = jnp.maximum(_dot(xb, wp_ref[...]) + bp_ref[...], 0.0)
  deg = degp_ref[0, :, 0] + degp_ref[1, :, 0] + 1.0
  dis = lax.rsqrt(deg)[:, None]
  h_ref[...] = h
  y0_ref[...] = _dot(h, w0_ref[...]) * dis
  dis_ref[...] = dis


def _tc0(x_p, Wp, bp, W0f, degp):
  return pl.pallas_call(
      _tc0_body,
      grid=(GRID,),
      in_specs=[
          pl.BlockSpec((BLK, D), lambda i: (i, 0)),
          pl.BlockSpec((D, D), lambda i: (0, 0)),
          pl.BlockSpec((1, D), lambda i: (0, 0)),
          pl.BlockSpec((D, D), lambda i: (0, 0)),
          pl.BlockSpec((NC, BLK, D), lambda i: (0, i, 0)),
      ],
      out_specs=[
          pl.BlockSpec((BLK, D), lambda i: (i, 0)),
          pl.BlockSpec((BLK, D), lambda i: (i, 0)),
          pl.BlockSpec((BLK, 1), lambda i: (i, 0)),
      ],
      out_shape=[
          jax.ShapeDtypeStruct((N_PAD, D), jnp.float32),
          jax.ShapeDtypeStruct((N_PAD, D), jnp.float32),
          jax.ShapeDtypeStruct((N_PAD, 1), jnp.float32),
      ],
  )(x_p, Wp, bp, W0f, degp)


def _tcmid_body(S_ref, yp_ref, hp_ref, dis_ref, w_ref, bf_ref,
                h_ref, y_ref):
  dis = dis_ref[...]
  t = (S_ref[0] + S_ref[1] + yp_ref[...]) * dis + bf_ref[...]
  h = jnp.maximum(t, 0.0) + hp_ref[...]
  h_ref[...] = h
  y_ref[...] = _dot(h, w_ref[...]) * dis


def _tcmid(S, y_prev, h_prev, dis, Wf, bf):
  return pl.pallas_call(
      _tcmid_body,
      grid=(GRID,),
      in_specs=[
          pl.BlockSpec((NC, BLK, D), lambda i: (0, i, 0)),
          pl.BlockSpec((BLK, D), lambda i: (i, 0)),
          pl.BlockSpec((BLK, D), lambda i: (i, 0)),
          pl.BlockSpec((BLK, 1), lambda i: (i, 0)),
          pl.BlockSpec((D, D), lambda i: (0, 0)),
          pl.BlockSpec((1, D), lambda i: (0, 0)),
      ],
      out_specs=[
          pl.BlockSpec((BLK, D), lambda i: (i, 0)),
          pl.BlockSpec((BLK, D), lambda i: (i, 0)),
      ],
      out_shape=[
          jax.ShapeDtypeStruct((N_PAD, D), jnp.float32),
          jax.ShapeDtypeStruct((N_PAD, D), jnp.float32),
      ],
  )(S, y_prev, h_prev, dis, Wf, bf)


def _tcfin_body(S_ref, yp_ref, dis_ref, bf_ref, out_ref):
  out_ref[...] = (S_ref[0] + S_ref[1] + yp_ref[...]) * dis_ref[...] \
      + bf_ref[...]


def _tcfin(S, y2, dis, bf):
  return pl.pallas_call(
      _tcfin_body,
      grid=(GRID,),
      in_specs=[
          pl.BlockSpec((NC, BLK, D), lambda i: (0, i, 0)),
          pl.BlockSpec((BLK, D), lambda i: (i, 0)),
          pl.BlockSpec((BLK, 1), lambda i: (i, 0)),
          pl.BlockSpec((1, D), lambda i: (0, 0)),
      ],
      out_specs=pl.BlockSpec((BLK, D), lambda i: (i, 0)),
      out_shape=jax.ShapeDtypeStruct((N_PAD, D), jnp.float32),
  )(S, y2, dis, bf)


def kernel(x, edge_index, edge_attr, Wp, bp, W0, b0, W1, b1, W2, b2,
           g0, be0, g1, be1, g2, be2):
  del edge_attr
  f32 = jnp.float32
  c_bn = 1.0 / jnp.sqrt(jnp.asarray(1.0 + EPS, f32))

  W0f = W0 * (c_bn * g0)[None, :]
  W1f = W1 * (c_bn * g1)[None, :]
  W2f = W2 * (c_bn * g2)[None, :]
  b0f = (b0 * c_bn * g0 + be0)[None, :]
  b1f = (b1 * c_bn * g1 + be1)[None, :]
  b2f = (b2 * c_bn * g2 + be2)[None, :]
  bp2 = bp[None, :]

  x_p = jnp.concatenate([x, jnp.zeros((N_PAD - N, D), f32)], axis=0)
  npad = E_PAD - E
  pad_iota = lax.iota(jnp.int32, npad)
  pad_src = pad_iota % N
  pad_dst = N + (pad_iota % (N_PAD - N))
  src_p = jnp.concatenate([edge_index[0], pad_src])
  dst_p = jnp.concatenate([edge_index[1], pad_dst])

  zeros128 = jnp.zeros((ROWS_W, D), f32)
  ones128 = jnp.ones((CH, D), f32)

  degp = _deg_kernel()(dst_p, zeros128, ones128)
  h, y0, dis = _tc0(x_p, Wp, bp2, W0f, degp)

  S0 = _edge_kernel()(y0, src_p, dst_p, zeros128)
  h1, y1 = _tcmid(S0, y0, h, dis, W1f, b0f)

  S1 = _edge_kernel()(y1, src_p, dst_p, zeros128)
  h2, y2 = _tcmid(S1, y1, h1, dis, W2f, b1f)

  S2 = _edge_kernel()(y2, src_p, dst_p, zeros128)
  out = _tcfin(S2, y2, dis, b2f)
  return out[:N]


# speedup vs baseline: 20.6084x; 1.7377x over previous
import functools
import jax
import jax.numpy as jnp
from jax import lax
from jax.experimental import pallas as pl
from jax.experimental.pallas import tpu as pltpu
from jax.experimental.pallas import tpu_sc as plsc

N = 10000
E = 320000
D = 128
EPS = 1e-5

NC = 2
NS = 16
NW = NC * NS

CH = 128
NCH = 80
E_W = CH * NCH
E_PAD = NW * E_W
GRP = 2
GCH = NCH // GRP

N_PAD = 10240
ROWS_W = N_PAD // NS
BLK = 256
GRID = N_PAD // BLK

@functools.cache
def _mesh():
  return plsc.VectorSubcoreMesh(
      core_axis_name="c", subcore_axis_name="s", num_cores=NC, num_subcores=NS)


DEG_Q = 8

def _deg_body(dst_hbm, zeros128, ones128, out_hbm, dstb, onesv, acc, sem):
  c = lax.axis_index("c")
  s = lax.axis_index("s")
  w = c * NS + s
  pltpu.sync_copy(zeros128, acc.at[pl.ds(s * ROWS_W, ROWS_W)])
  pltpu.sync_copy(ones128, onesv)
  pltpu.sync_copy(dst_hbm.at[pl.ds(w * NCH, NCH)], dstb)
  plsc.subcore_barrier()

  def chunk(j, _):
    @pl.when(j >= DEG_Q)
    def _w():
      pltpu.make_async_copy(onesv, acc.at[pl.ds(0, CH)], sem).wait()
    pltpu.async_copy(onesv, acc.at[dstb.at[j]], sem, add=True)
    return _

  lax.fori_loop(0, NCH, chunk, 0)
  for _ in range(DEG_Q):
    pltpu.make_async_copy(onesv, acc.at[pl.ds(0, CH)], sem).wait()
  plsc.subcore_barrier()
  pltpu.sync_copy(acc.at[pl.ds(s * ROWS_W, ROWS_W)],
                  out_hbm.at[c, pl.ds(s * ROWS_W, ROWS_W)])


@functools.cache
def _deg_kernel():
  return pl.kernel(
      _deg_body,
      out_type=jax.ShapeDtypeStruct((NC, N_PAD, D), jnp.float32),
      mesh=_mesh(),
      scratch_types=[
          pltpu.VMEM((NCH, CH), jnp.int32),
          pltpu.VMEM((CH, D), jnp.float32),
          pltpu.VMEM_SHARED((N_PAD, D), jnp.float32),
          pltpu.SemaphoreType.DMA,
      ],
  )


def _edge_body(y_hbm, src_hbm, dst_hbm, zeros128, out_hbm,
               srcb, dstb, r0, r1, acc, g0, g1):
  c = lax.axis_index("c")
  s = lax.axis_index("s")
  w = c * NS + s
  rows = [r0, r1]
  gs = [g0, g1]
  pltpu.sync_copy(zeros128, acc.at[pl.ds(s * ROWS_W, ROWS_W)])
  plsc.subcore_barrier()

  for G in range(GRP):
    base = w * NCH + G * GCH
    pltpu.sync_copy(src_hbm.at[pl.ds(base, GCH)], srcb)
    pltpu.sync_copy(dst_hbm.at[pl.ds(base, GCH)], dstb)
    pltpu.async_copy(y_hbm.at[srcb.at[0]], rows[0], gs[0])

    def chunk2(g, carry):
      for b in range(2):
        j = g * 2 + b
        nb = (b + 1) % 2

        @pl.when(j + 1 < GCH)
        def _gg():
          pltpu.async_copy(y_hbm.at[srcb.at[j + 1]], rows[nb], gs[nb])

        pltpu.make_async_copy(y_hbm.at[pl.ds(0, CH)], rows[b], gs[b]).wait()
        pltpu.sync_copy(rows[b], acc.at[dstb.at[j]], add=True)
      return carry

    lax.fori_loop(0, GCH // 2, chunk2, 0)
  plsc.subcore_barrier()
  pltpu.sync_copy(acc.at[pl.ds(s * ROWS_W, ROWS_W)],
                  out_hbm.at[c, pl.ds(s * ROWS_W, ROWS_W)])


@functools.cache
def _edge_kernel():
  return pl.kernel(
      _edge_body,
      out_type=jax.ShapeDtypeStruct((NC, N_PAD, D), jnp.float32),
      mesh=_mesh(),
      scratch_types=[
          pltpu.VMEM((GCH, CH), jnp.int32),
          pltpu.VMEM((GCH, CH), jnp.int32),
          pltpu.VMEM((CH, D), jnp.float32),
          pltpu.VMEM((CH, D), jnp.float32),
          pltpu.VMEM_SHARED((N_PAD, D), jnp.float32),
      ] + [pltpu.SemaphoreType.DMA] * 2,
  )


def _dot(a, b):
  return jax.lax.dot_general(a, b, (((1,), (0,)), ((), ())),
                             precision=lax.Precision.HIGHEST,
                             preferred_element_type=jnp.float32)


def _tc0_body(x_ref, wp_ref, bp_ref, w0_ref, degp_ref, h_ref, y0_ref, dis_ref):
  xb = x_ref[...]
  h = jnp.maximum(_dot(xb, wp_ref[...]) + bp_ref[...], 0.0)
  deg = degp_ref[0, :, 0] + degp_ref[1, :, 0] + 1.0
  dis = lax.rsqrt(deg)[:, None]
  h_ref[...] = h
  y0_ref[...] = _dot(h, w0_ref[...]) * dis
  dis_ref[...] = dis


def _tc0(x_p, Wp, bp, W0f, degp):
  return pl.pallas_call(
      _tc0_body,
      grid=(GRID,),
      in_specs=[
          pl.BlockSpec((BLK, D), lambda i: (i, 0)),
          pl.BlockSpec((D, D), lambda i: (0, 0)),
          pl.BlockSpec((1, D), lambda i: (0, 0)),
          pl.BlockSpec((D, D), lambda i: (0, 0)),
          pl.BlockSpec((NC, BLK, D), lambda i: (0, i, 0)),
      ],
      out_specs=[
          pl.BlockSpec((BLK, D), lambda i: (i, 0)),
          pl.BlockSpec((BLK, D), lambda i: (i, 0)),
          pl.BlockSpec((BLK, 1), lambda i: (i, 0)),
      ],
      out_shape=[
          jax.ShapeDtypeStruct((N_PAD, D), jnp.float32),
          jax.ShapeDtypeStruct((N_PAD, D), jnp.float32),
          jax.ShapeDtypeStruct((N_PAD, 1), jnp.float32),
      ],
  )(x_p, Wp, bp, W0f, degp)


def _tcmid_body(S_ref, yp_ref, hp_ref, dis_ref, w_ref, bf_ref,
                h_ref, y_ref):
  dis = dis_ref[...]
  t = (S_ref[0] + S_ref[1] + yp_ref[...]) * dis + bf_ref[...]
  h = jnp.maximum(t, 0.0) + hp_ref[...]
  h_ref[...] = h
  y_ref[...] = _dot(h, w_ref[...]) * dis


def _tcmid(S, y_prev, h_prev, dis, Wf, bf):
  return pl.pallas_call(
      _tcmid_body,
      grid=(GRID,),
      in_specs=[
          pl.BlockSpec((NC, BLK, D), lambda i: (0, i, 0)),
          pl.BlockSpec((BLK, D), lambda i: (i, 0)),
          pl.BlockSpec((BLK, D), lambda i: (i, 0)),
          pl.BlockSpec((BLK, 1), lambda i: (i, 0)),
          pl.BlockSpec((D, D), lambda i: (0, 0)),
          pl.BlockSpec((1, D), lambda i: (0, 0)),
      ],
      out_specs=[
          pl.BlockSpec((BLK, D), lambda i: (i, 0)),
          pl.BlockSpec((BLK, D), lambda i: (i, 0)),
      ],
      out_shape=[
          jax.ShapeDtypeStruct((N_PAD, D), jnp.float32),
          jax.ShapeDtypeStruct((N_PAD, D), jnp.float32),
      ],
  )(S, y_prev, h_prev, dis, Wf, bf)


def _tcfin_body(S_ref, yp_ref, dis_ref, bf_ref, out_ref):
  out_ref[...] = (S_ref[0] + S_ref[1] + yp_ref[...]) * dis_ref[...] \
      + bf_ref[...]


def _tcfin(S, y2, dis, bf):
  return pl.pallas_call(
      _tcfin_body,
      grid=(GRID,),
      in_specs=[
          pl.BlockSpec((NC, BLK, D), lambda i: (0, i, 0)),
          pl.BlockSpec((BLK, D), lambda i: (i, 0)),
          pl.BlockSpec((BLK, 1), lambda i: (i, 0)),
          pl.BlockSpec((1, D), lambda i: (0, 0)),
      ],
      out_specs=pl.BlockSpec((BLK, D), lambda i: (i, 0)),
      out_shape=jax.ShapeDtypeStruct((N_PAD, D), jnp.float32),
  )(S, y2, dis, bf)


def kernel(x, edge_index, edge_attr, Wp, bp, W0, b0, W1, b1, W2, b2,
           g0, be0, g1, be1, g2, be2):
  del edge_attr
  f32 = jnp.float32
  c_bn = 1.0 / jnp.sqrt(jnp.asarray(1.0 + EPS, f32))

  W0f = W0 * (c_bn * g0)[None, :]
  W1f = W1 * (c_bn * g1)[None, :]
  W2f = W2 * (c_bn * g2)[None, :]
  b0f = (b0 * c_bn * g0 + be0)[None, :]
  b1f = (b1 * c_bn * g1 + be1)[None, :]
  b2f = (b2 * c_bn * g2 + be2)[None, :]
  bp2 = bp[None, :]

  x_p = jnp.concatenate([x, jnp.zeros((N_PAD - N, D), f32)], axis=0)
  npad = E_PAD - E
  pad_iota = lax.iota(jnp.int32, npad)
  pad_src = pad_iota % N
  pad_dst = N + (pad_iota % (N_PAD - N))
  src_p = jnp.concatenate([edge_index[0], pad_src]).reshape(NW * NCH, CH)
  dst_p = jnp.concatenate([edge_index[1], pad_dst]).reshape(NW * NCH, CH)

  zeros128 = jnp.zeros((ROWS_W, D), f32)
  ones128 = jnp.ones((CH, D), f32)

  degp = _deg_kernel()(dst_p, zeros128, ones128)
  h, y0, dis = _tc0(x_p, Wp, bp2, W0f, degp)

  S0 = _edge_kernel()(y0, src_p, dst_p, zeros128)
  h1, y1 = _tcmid(S0, y0, h, dis, W1f, b0f)

  S1 = _edge_kernel()(y1, src_p, dst_p, zeros128)
  h2, y2 = _tcmid(S1, y1, h1, dis, W2f, b1f)

  S2 = _edge_kernel()(y2, src_p, dst_p, zeros128)
  out = _tcfin(S2, y2, dis, b2f)
  return out[:N]


# speedup vs baseline: 21.0390x; 1.0209x over previous
import functools
import jax
import jax.numpy as jnp
from jax import lax
from jax.experimental import pallas as pl
from jax.experimental.pallas import tpu as pltpu
from jax.experimental.pallas import tpu_sc as plsc

N = 10000
E = 320000
D = 128
EPS = 1e-5

NC = 2
NS = 16
NW = NC * NS

CH = 128
NCH = 80
E_W = CH * NCH
E_PAD = NW * E_W

ECH = 64
ENCH = E_W // ECH
EGRP = 2
EGCH = ENCH // EGRP
NB = 4

N_PAD = 10240
ROWS_W = N_PAD // NS
BLK = 256
GRID = N_PAD // BLK

@functools.cache
def _mesh():
  return plsc.VectorSubcoreMesh(
      core_axis_name="c", subcore_axis_name="s", num_cores=NC, num_subcores=NS)


DEG_Q = 8

def _deg_body(dst_hbm, zeros128, ones128, out_hbm, dstb, onesv, acc, sem):
  c = lax.axis_index("c")
  s = lax.axis_index("s")
  w = c * NS + s
  pltpu.sync_copy(zeros128, acc.at[pl.ds(s * ROWS_W, ROWS_W)])
  pltpu.sync_copy(ones128, onesv)
  pltpu.sync_copy(dst_hbm.at[pl.ds(w * NCH, NCH)], dstb)
  plsc.subcore_barrier()

  def chunk(j, _):
    @pl.when(j >= DEG_Q)
    def _w():
      pltpu.make_async_copy(onesv, acc.at[pl.ds(0, CH)], sem).wait()
    pltpu.async_copy(onesv, acc.at[dstb.at[j]], sem, add=True)
    return _

  lax.fori_loop(0, NCH, chunk, 0)
  for _ in range(DEG_Q):
    pltpu.make_async_copy(onesv, acc.at[pl.ds(0, CH)], sem).wait()
  plsc.subcore_barrier()
  pltpu.sync_copy(acc.at[pl.ds(s * ROWS_W, ROWS_W)],
                  out_hbm.at[c, pl.ds(s * ROWS_W, ROWS_W)])


@functools.cache
def _deg_kernel():
  return pl.kernel(
      _deg_body,
      out_type=jax.ShapeDtypeStruct((NC, N_PAD, D), jnp.float32),
      mesh=_mesh(),
      scratch_types=[
          pltpu.VMEM((NCH, CH), jnp.int32),
          pltpu.VMEM((CH, D), jnp.float32),
          pltpu.VMEM_SHARED((N_PAD, D), jnp.float32),
          pltpu.SemaphoreType.DMA,
      ],
  )


def _edge_body(y_hbm, src_hbm, dst_hbm, zeros128, out_hbm,
               srcb, dstb, r0, r1, r2, r3, acc,
               g0, g1, g2, g3, s0, s1, s2, s3):
  c = lax.axis_index("c")
  s = lax.axis_index("s")
  w = c * NS + s
  rows = [r0, r1, r2, r3]
  gs = [g0, g1, g2, g3]
  ss = [s0, s1, s2, s3]
  pltpu.sync_copy(zeros128, acc.at[pl.ds(s * ROWS_W, ROWS_W)])
  plsc.subcore_barrier()

  def _src_idx(j):
    return srcb.at[j // 2, pl.ds((j % 2) * ECH, ECH)]

  for G in range(EGRP):
    pltpu.sync_copy(src_hbm.at[pl.ds(w * NCH + G * (EGCH // 2), EGCH // 2)],
                    srcb)
    pltpu.sync_copy(dst_hbm.at[pl.ds(w * ENCH + G * EGCH, EGCH)], dstb)
    for b in range(2):
      pltpu.async_copy(y_hbm.at[_src_idx(b)], rows[b], gs[b])

    def chunk4(g, carry):
      for b in range(NB):
        j = g * NB + b
        jb2 = (b + 2) % NB

        @pl.when(j >= 2)
        def _ws():
          pltpu.make_async_copy(rows[jb2], acc.at[pl.ds(0, ECH)],
                                ss[jb2]).wait()

        @pl.when(j + 2 < EGCH)
        def _gg():
          pltpu.async_copy(y_hbm.at[_src_idx(j + 2)], rows[jb2], gs[jb2])

        pltpu.make_async_copy(y_hbm.at[pl.ds(0, ECH)], rows[b], gs[b]).wait()
        pltpu.async_copy(rows[b], acc.at[dstb.at[j]], ss[b], add=True)
      return carry

    lax.fori_loop(0, EGCH // NB, chunk4, 0)
    for j in (EGCH - 2, EGCH - 1):
      pltpu.make_async_copy(rows[j % NB], acc.at[pl.ds(0, ECH)],
                            ss[j % NB]).wait()
  plsc.subcore_barrier()
  pltpu.sync_copy(acc.at[pl.ds(s * ROWS_W, ROWS_W)],
                  out_hbm.at[c, pl.ds(s * ROWS_W, ROWS_W)])


@functools.cache
def _edge_kernel():
  return pl.kernel(
      _edge_body,
      out_type=jax.ShapeDtypeStruct((NC, N_PAD, D), jnp.float32),
      mesh=_mesh(),
      scratch_types=[
          pltpu.VMEM((EGCH // 2, CH), jnp.int32),
          pltpu.VMEM((EGCH, ECH), jnp.int32),
          pltpu.VMEM((ECH, D), jnp.float32),
          pltpu.VMEM((ECH, D), jnp.float32),
          pltpu.VMEM((ECH, D), jnp.float32),
          pltpu.VMEM((ECH, D), jnp.float32),
          pltpu.VMEM_SHARED((N_PAD, D), jnp.float32),
      ] + [pltpu.SemaphoreType.DMA] * 8,
  )


def _dot(a, b):
  return jax.lax.dot_general(a, b, (((1,), (0,)), ((), ())),
                             precision=lax.Precision.HIGHEST,
                             preferred_element_type=jnp.float32)


def _tc0_body(x_ref, wp_ref, bp_ref, w0_ref, degp_ref, h_ref, y0_ref, dis_ref):
  xb = x_ref[...]
  h = jnp.maximum(_dot(xb, wp_ref[...]) + bp_ref[...], 0.0)
  deg = degp_ref[0, :, 0] + degp_ref[1, :, 0] + 1.0
  dis = lax.rsqrt(deg)[:, None]
  h_ref[...] = h
  y0_ref[...] = _dot(h, w0_ref[...]) * dis
  dis_ref[...] = dis


def _tc0(x_p, Wp, bp, W0f, degp):
  return pl.pallas_call(
      _tc0_body,
      grid=(GRID,),
      in_specs=[
          pl.BlockSpec((BLK, D), lambda i: (i, 0)),
          pl.BlockSpec((D, D), lambda i: (0, 0)),
          pl.BlockSpec((1, D), lambda i: (0, 0)),
          pl.BlockSpec((D, D), lambda i: (0, 0)),
          pl.BlockSpec((NC, BLK, D), lambda i: (0, i, 0)),
      ],
      out_specs=[
          pl.BlockSpec((BLK, D), lambda i: (i, 0)),
          pl.BlockSpec((BLK, D), lambda i: (i, 0)),
          pl.BlockSpec((BLK, 1), lambda i: (i, 0)),
      ],
      out_shape=[
          jax.ShapeDtypeStruct((N_PAD, D), jnp.float32),
          jax.ShapeDtypeStruct((N_PAD, D), jnp.float32),
          jax.ShapeDtypeStruct((N_PAD, 1), jnp.float32),
      ],
  )(x_p, Wp, bp, W0f, degp)


def _tcmid_body(S_ref, yp_ref, hp_ref, dis_ref, w_ref, bf_ref,
                h_ref, y_ref):
  dis = dis_ref[...]
  t = (S_ref[0] + S_ref[1] + yp_ref[...]) * dis + bf_ref[...]
  h = jnp.maximum(t, 0.0) + hp_ref[...]
  h_ref[...] = h
  y_ref[...] = _dot(h, w_ref[...]) * dis


def _tcmid(S, y_prev, h_prev, dis, Wf, bf):
  return pl.pallas_call(
      _tcmid_body,
      grid=(GRID,),
      in_specs=[
          pl.BlockSpec((NC, BLK, D), lambda i: (0, i, 0)),
          pl.BlockSpec((BLK, D), lambda i: (i, 0)),
          pl.BlockSpec((BLK, D), lambda i: (i, 0)),
          pl.BlockSpec((BLK, 1), lambda i: (i, 0)),
          pl.BlockSpec((D, D), lambda i: (0, 0)),
          pl.BlockSpec((1, D), lambda i: (0, 0)),
      ],
      out_specs=[
          pl.BlockSpec((BLK, D), lambda i: (i, 0)),
          pl.BlockSpec((BLK, D), lambda i: (i, 0)),
      ],
      out_shape=[
          jax.ShapeDtypeStruct((N_PAD, D), jnp.float32),
          jax.ShapeDtypeStruct((N_PAD, D), jnp.float32),
      ],
  )(S, y_prev, h_prev, dis, Wf, bf)


def _tcfin_body(S_ref, yp_ref, dis_ref, bf_ref, out_ref):
  out_ref[...] = (S_ref[0] + S_ref[1] + yp_ref[...]) * dis_ref[...] \
      + bf_ref[...]


def _tcfin(S, y2, dis, bf):
  return pl.pallas_call(
      _tcfin_body,
      grid=(GRID,),
      in_specs=[
          pl.BlockSpec((NC, BLK, D), lambda i: (0, i, 0)),
          pl.BlockSpec((BLK, D), lambda i: (i, 0)),
          pl.BlockSpec((BLK, 1), lambda i: (i, 0)),
          pl.BlockSpec((1, D), lambda i: (0, 0)),
      ],
      out_specs=pl.BlockSpec((BLK, D), lambda i: (i, 0)),
      out_shape=jax.ShapeDtypeStruct((N_PAD, D), jnp.float32),
  )(S, y2, dis, bf)


def kernel(x, edge_index, edge_attr, Wp, bp, W0, b0, W1, b1, W2, b2,
           g0, be0, g1, be1, g2, be2):
  del edge_attr
  f32 = jnp.float32
  c_bn = 1.0 / jnp.sqrt(jnp.asarray(1.0 + EPS, f32))

  W0f = W0 * (c_bn * g0)[None, :]
  W1f = W1 * (c_bn * g1)[None, :]
  W2f = W2 * (c_bn * g2)[None, :]
  b0f = (b0 * c_bn * g0 + be0)[None, :]
  b1f = (b1 * c_bn * g1 + be1)[None, :]
  b2f = (b2 * c_bn * g2 + be2)[None, :]
  bp2 = bp[None, :]

  x_p = jnp.concatenate([x, jnp.zeros((N_PAD - N, D), f32)], axis=0)
  npad = E_PAD - E
  pad_iota = lax.iota(jnp.int32, npad)
  pad_src = pad_iota % N
  pad_dst = N + (pad_iota % (N_PAD - N))
  src_flat = jnp.concatenate([edge_index[0], pad_src])
  dst_flat = jnp.concatenate([edge_index[1], pad_dst])
  src_p = src_flat.reshape(NW * NCH, CH)
  dst_p = dst_flat.reshape(NW * ENCH, ECH)
  dst_deg = dst_flat.reshape(NW * NCH, CH)

  zeros128 = jnp.zeros((ROWS_W, D), f32)
  ones128 = jnp.ones((CH, D), f32)

  degp = _deg_kernel()(dst_deg, zeros128, ones128)
  h, y0, dis = _tc0(x_p, Wp, bp2, W0f, degp)

  S0 = _edge_kernel()(y0, src_p, dst_p, zeros128)
  h1, y1 = _tcmid(S0, y0, h, dis, W1f, b0f)

  S1 = _edge_kernel()(y1, src_p, dst_p, zeros128)
  h2, y2 = _tcmid(S1, y1, h1, dis, W2f, b1f)

  S2 = _edge_kernel()(y2, src_p, dst_p, zeros128)
  out = _tcfin(S2, y2, dis, b2f)
  return out[:N]


# speedup vs baseline: 23.4034x; 1.1124x over previous
import functools
import jax
import jax.numpy as jnp
from jax import lax
from jax.experimental import pallas as pl
from jax.experimental.pallas import tpu as pltpu
from jax.experimental.pallas import tpu_sc as plsc

N = 10000
E = 320000
D = 128
EPS = 1e-5

NC = 2
NS = 16
NW = NC * NS

CH = 128
NCH = 80
E_W = CH * NCH
E_PAD = NW * E_W

ECH = 64
ENCH = E_W // ECH
EGRP = 2
EGCH = ENCH // EGRP
NB = 4

N_PAD = 10240
ROWS_W = N_PAD // NS
BLK = 512
GRID = N_PAD // BLK

@functools.cache
def _mesh():
  return plsc.VectorSubcoreMesh(
      core_axis_name="c", subcore_axis_name="s", num_cores=NC, num_subcores=NS)


DEG_Q = 8

def _deg_body(dst_hbm, zeros128, ones128, out_hbm, dstb, onesv, acc, sem):
  c = lax.axis_index("c")
  s = lax.axis_index("s")
  w = c * NS + s
  pltpu.sync_copy(zeros128, acc.at[pl.ds(s * ROWS_W, ROWS_W)])
  pltpu.sync_copy(ones128, onesv)
  pltpu.sync_copy(dst_hbm.at[pl.ds(w * NCH, NCH)], dstb)
  plsc.subcore_barrier()

  def chunk(j, _):
    @pl.when(j >= DEG_Q)
    def _w():
      pltpu.make_async_copy(onesv, acc.at[pl.ds(0, CH)], sem).wait()
    pltpu.async_copy(onesv, acc.at[dstb.at[j]], sem, add=True)
    return _

  lax.fori_loop(0, NCH, chunk, 0)
  for _ in range(DEG_Q):
    pltpu.make_async_copy(onesv, acc.at[pl.ds(0, CH)], sem).wait()
  plsc.subcore_barrier()
  pltpu.sync_copy(acc.at[pl.ds(s * ROWS_W, ROWS_W)],
                  out_hbm.at[c, pl.ds(s * ROWS_W, ROWS_W)])


@functools.cache
def _deg_kernel():
  return pl.kernel(
      _deg_body,
      out_type=jax.ShapeDtypeStruct((NC, N_PAD, D), jnp.float32),
      mesh=_mesh(),
      scratch_types=[
          pltpu.VMEM((NCH, CH), jnp.int32),
          pltpu.VMEM((CH, D), jnp.float32),
          pltpu.VMEM_SHARED((N_PAD, D), jnp.float32),
          pltpu.SemaphoreType.DMA,
      ],
  )


def _edge_body(y_hbm, src_hbm, dst_hbm, zeros128, out_hbm,
               srcb, dstb, r0, r1, r2, r3, acc,
               g0, g1, g2, g3, s0, s1, s2, s3):
  c = lax.axis_index("c")
  s = lax.axis_index("s")
  w = c * NS + s
  rows = [r0, r1, r2, r3]
  gs = [g0, g1, g2, g3]
  ss = [s0, s1, s2, s3]
  pltpu.sync_copy(zeros128, acc.at[pl.ds(s * ROWS_W, ROWS_W)])
  plsc.subcore_barrier()

  def _src_idx(j):
    return srcb.at[j // 2, pl.ds((j % 2) * ECH, ECH)]

  for G in range(EGRP):
    pltpu.sync_copy(src_hbm.at[pl.ds(w * NCH + G * (EGCH // 2), EGCH // 2)],
                    srcb)
    pltpu.sync_copy(dst_hbm.at[pl.ds(w * ENCH + G * EGCH, EGCH)], dstb)
    for b in range(2):
      pltpu.async_copy(y_hbm.at[_src_idx(b)], rows[b], gs[b])

    def chunk4(g, carry):
      for b in range(NB):
        j = g * NB + b
        jb2 = (b + 2) % NB

        @pl.when(j >= 2)
        def _ws():
          pltpu.make_async_copy(rows[jb2], acc.at[pl.ds(0, ECH)],
                                ss[jb2]).wait()

        @pl.when(j + 2 < EGCH)
        def _gg():
          pltpu.async_copy(y_hbm.at[_src_idx(j + 2)], rows[jb2], gs[jb2])

        pltpu.make_async_copy(y_hbm.at[pl.ds(0, ECH)], rows[b], gs[b]).wait()
        pltpu.async_copy(rows[b], acc.at[dstb.at[j]], ss[b], add=True)
      return carry

    lax.fori_loop(0, EGCH // NB, chunk4, 0)
    for j in (EGCH - 2, EGCH - 1):
      pltpu.make_async_copy(rows[j % NB], acc.at[pl.ds(0, ECH)],
                            ss[j % NB]).wait()
  plsc.subcore_barrier()
  pltpu.sync_copy(acc.at[pl.ds(s * ROWS_W, ROWS_W)],
                  out_hbm.at[c, pl.ds(s * ROWS_W, ROWS_W)])


@functools.cache
def _edge_kernel():
  return pl.kernel(
      _edge_body,
      out_type=jax.ShapeDtypeStruct((NC, N_PAD, D), jnp.float32),
      mesh=_mesh(),
      scratch_types=[
          pltpu.VMEM((EGCH // 2, CH), jnp.int32),
          pltpu.VMEM((EGCH, ECH), jnp.int32),
          pltpu.VMEM((ECH, D), jnp.float32),
          pltpu.VMEM((ECH, D), jnp.float32),
          pltpu.VMEM((ECH, D), jnp.float32),
          pltpu.VMEM((ECH, D), jnp.float32),
          pltpu.VMEM_SHARED((N_PAD, D), jnp.float32),
      ] + [pltpu.SemaphoreType.DMA] * 8,
  )


def _dot(a, b):
  return jax.lax.dot_general(a, b, (((1,), (0,)), ((), ())),
                             precision=lax.Precision.HIGHEST,
                             preferred_element_type=jnp.float32)


def _tch_body(x_ref, wp_ref, bp_ref, h_ref):
  h_ref[...] = jnp.maximum(_dot(x_ref[...], wp_ref[...]) + bp_ref[...], 0.0)


def _tch(x, Wp, bp):
  return pl.pallas_call(
      _tch_body,
      grid=(GRID,),
      in_specs=[
          pl.BlockSpec((BLK, D), lambda i: (i, 0)),
          pl.BlockSpec((D, D), lambda i: (0, 0)),
          pl.BlockSpec((1, D), lambda i: (0, 0)),
      ],
      out_specs=pl.BlockSpec((BLK, D), lambda i: (i, 0)),
      out_shape=jax.ShapeDtypeStruct((N_PAD, D), jnp.float32),
  )(x, Wp, bp)


def _tcy0_body(h_ref, w0_ref, degp_ref, y0_ref, dis_ref):
  deg = degp_ref[0, :, 0] + degp_ref[1, :, 0] + 1.0
  dis = lax.rsqrt(deg)[:, None]
  y0_ref[...] = _dot(h_ref[...], w0_ref[...]) * dis
  dis_ref[...] = dis


def _tcy0(h, W0f, degp):
  return pl.pallas_call(
      _tcy0_body,
      grid=(GRID,),
      in_specs=[
          pl.BlockSpec((BLK, D), lambda i: (i, 0)),
          pl.BlockSpec((D, D), lambda i: (0, 0)),
          pl.BlockSpec((NC, BLK, D), lambda i: (0, i, 0)),
      ],
      out_specs=[
          pl.BlockSpec((BLK, D), lambda i: (i, 0)),
          pl.BlockSpec((BLK, 1), lambda i: (i, 0)),
      ],
      out_shape=[
          jax.ShapeDtypeStruct((N_PAD, D), jnp.float32),
          jax.ShapeDtypeStruct((N_PAD, 1), jnp.float32),
      ],
  )(h, W0f, degp)


def _tcmid_body(S_ref, yp_ref, hp_ref, dis_ref, w_ref, bf_ref,
                h_ref, y_ref):
  dis = dis_ref[...]
  t = (S_ref[0] + S_ref[1] + yp_ref[...]) * dis + bf_ref[...]
  h = jnp.maximum(t, 0.0) + hp_ref[...]
  h_ref[...] = h
  y_ref[...] = _dot(h, w_ref[...]) * dis


def _tcmid(S, y_prev, h_prev, dis, Wf, bf):
  return pl.pallas_call(
      _tcmid_body,
      grid=(GRID,),
      in_specs=[
          pl.BlockSpec((NC, BLK, D), lambda i: (0, i, 0)),
          pl.BlockSpec((BLK, D), lambda i: (i, 0)),
          pl.BlockSpec((BLK, D), lambda i: (i, 0)),
          pl.BlockSpec((BLK, 1), lambda i: (i, 0)),
          pl.BlockSpec((D, D), lambda i: (0, 0)),
          pl.BlockSpec((1, D), lambda i: (0, 0)),
      ],
      out_specs=[
          pl.BlockSpec((BLK, D), lambda i: (i, 0)),
          pl.BlockSpec((BLK, D), lambda i: (i, 0)),
      ],
      out_shape=[
          jax.ShapeDtypeStruct((N_PAD, D), jnp.float32),
          jax.ShapeDtypeStruct((N_PAD, D), jnp.float32),
      ],
  )(S, y_prev, h_prev, dis, Wf, bf)


def _tcfin_body(S_ref, yp_ref, dis_ref, bf_ref, out_ref):
  out_ref[...] = (S_ref[0] + S_ref[1] + yp_ref[...]) * dis_ref[...] \
      + bf_ref[...]


def _tcfin(S, y2, dis, bf):
  return pl.pallas_call(
      _tcfin_body,
      grid=(GRID,),
      in_specs=[
          pl.BlockSpec((NC, BLK, D), lambda i: (0, i, 0)),
          pl.BlockSpec((BLK, D), lambda i: (i, 0)),
          pl.BlockSpec((BLK, 1), lambda i: (i, 0)),
          pl.BlockSpec((1, D), lambda i: (0, 0)),
      ],
      out_specs=pl.BlockSpec((BLK, D), lambda i: (i, 0)),
      out_shape=jax.ShapeDtypeStruct((N, D), jnp.float32),
  )(S, y2, dis, bf)


def kernel(x, edge_index, edge_attr, Wp, bp, W0, b0, W1, b1, W2, b2,
           g0, be0, g1, be1, g2, be2):
  del edge_attr
  f32 = jnp.float32
  c_bn = 1.0 / jnp.sqrt(jnp.asarray(1.0 + EPS, f32))

  W0f = W0 * (c_bn * g0)[None, :]
  W1f = W1 * (c_bn * g1)[None, :]
  W2f = W2 * (c_bn * g2)[None, :]
  b0f = (b0 * c_bn * g0 + be0)[None, :]
  b1f = (b1 * c_bn * g1 + be1)[None, :]
  b2f = (b2 * c_bn * g2 + be2)[None, :]
  bp2 = bp[None, :]

  npad = E_PAD - E
  pad_iota = lax.iota(jnp.int32, npad)
  pad_src = pad_iota % N
  pad_dst = N + (pad_iota % (N_PAD - N))
  src_flat = jnp.concatenate([edge_index[0], pad_src])
  dst_flat = jnp.concatenate([edge_index[1], pad_dst])
  src_p = src_flat.reshape(NW * NCH, CH)
  dst_p = dst_flat.reshape(NW * ENCH, ECH)
  dst_deg = dst_flat.reshape(NW * NCH, CH)

  zeros128 = jnp.zeros((ROWS_W, D), f32)
  ones128 = jnp.ones((CH, D), f32)

  degp = _deg_kernel()(dst_deg, zeros128, ones128)
  h = _tch(x, Wp, bp2)
  y0, dis = _tcy0(h, W0f, degp)

  S0 = _edge_kernel()(y0, src_p, dst_p, zeros128)
  h1, y1 = _tcmid(S0, y0, h, dis, W1f, b0f)

  S1 = _edge_kernel()(y1, src_p, dst_p, zeros128)
  h2, y2 = _tcmid(S1, y1, h1, dis, W2f, b1f)

  S2 = _edge_kernel()(y2, src_p, dst_p, zeros128)
  return _tcfin(S2, y2, dis, b2f)
